# baseline (speedup 1.0000x reference)
import jax
import jax.numpy as jnp
from jax.experimental import pallas as pl
from jax.experimental.pallas import tpu as pltpu


def _round_up(a, m):
    return ((a + m - 1) // m) * m


def _matmul_bias_kernel(x_ref, w_ref, b_ref, o_ref):
    xb = x_ref[...].astype(jnp.bfloat16)
    acc = jax.lax.dot_general(
        xb, w_ref[...],
        dimension_numbers=(((1,), (1,)), ((), ())),
        preferred_element_type=jnp.float32,
    )
    o_ref[...] = acc + b_ref[...]


def kernel(x, weight, bias, mask):
    B, K = x.shape
    V = weight.shape[0]
    out_dtype = x.dtype

    w_bf = (weight * mask[:, None]).astype(jnp.bfloat16)
    b_m = (bias * mask).astype(jnp.float32)[None, :]

    tb = min(512, _round_up(B, 8))
    Bp, Vp, Kp = _round_up(B, tb), _round_up(V, 128), _round_up(K, 128)
    if Bp != B or Kp != K:
        x = jnp.pad(x, ((0, Bp - B), (0, Kp - K)))
    if Vp != V or Kp != K:
        w_bf = jnp.pad(w_bf, ((0, Vp - V), (0, Kp - K)))
    if Vp != V:
        b_m = jnp.pad(b_m, ((0, 0), (0, Vp - V)))

    grid = (Bp // tb,)
    out = pl.pallas_call(
        _matmul_bias_kernel,
        out_shape=jax.ShapeDtypeStruct((Bp, Vp), jnp.float32),
        grid=grid,
        in_specs=[
            pl.BlockSpec((tb, Kp), lambda i: (i, 0)),
            pl.BlockSpec((Vp, Kp), lambda i: (0, 0)),
            pl.BlockSpec((1, Vp), lambda i: (0, 0)),
        ],
        out_specs=pl.BlockSpec((tb, Vp), lambda i: (i, 0)),
        compiler_params=pltpu.CompilerParams(
            dimension_semantics=("arbitrary",)),
    )(x, w_bf, b_m)

    if Bp != B or Vp != V:
        out = out[:B, :V]
    return out.astype(out_dtype)


# speedup vs baseline: 1.1124x; 1.1124x over previous
import jax
import jax.numpy as jnp
from jax.experimental import pallas as pl
from jax.experimental.pallas import tpu as pltpu


def _round_up(a, m):
    return ((a + m - 1) // m) * m


def _masked_linear_kernel(x_ref, w_ref, bm_ref, o_ref, wbf_ref):
    @pl.when(pl.program_id(0) == 0)
    def _cast_weight():
        wbf_ref[...] = w_ref[...].astype(jnp.bfloat16)

    xb = x_ref[...].astype(jnp.bfloat16)
    acc = jax.lax.dot_general(
        xb, wbf_ref[...],
        dimension_numbers=(((1,), (1,)), ((), ())),
        preferred_element_type=jnp.float32,
    )
    o_ref[...] = acc * bm_ref[1:2, :] + bm_ref[0:1, :]


def kernel(x, weight, bias, mask):
    B, K = x.shape
    V = weight.shape[0]
    out_dtype = x.dtype

    m_f = mask.astype(jnp.float32)
    bm = jnp.stack([bias.astype(jnp.float32) * m_f, m_f], axis=0)

    tb = min(512, _round_up(B, 8))
    Bp, Vp, Kp = _round_up(B, tb), _round_up(V, 128), _round_up(K, 128)
    if Bp != B or Kp != K:
        x = jnp.pad(x, ((0, Bp - B), (0, Kp - K)))
    if Vp != V or Kp != K:
        weight = jnp.pad(weight, ((0, Vp - V), (0, Kp - K)))
    if Vp != V:
        bm = jnp.pad(bm, ((0, 0), (0, Vp - V)))

    grid = (Bp // tb,)
    out = pl.pallas_call(
        _masked_linear_kernel,
        out_shape=jax.ShapeDtypeStruct((Bp, Vp), jnp.float32),
        grid=grid,
        in_specs=[
            pl.BlockSpec((tb, Kp), lambda i: (i, 0)),
            pl.BlockSpec((Vp, Kp), lambda i: (0, 0)),
            pl.BlockSpec((2, Vp), lambda i: (0, 0)),
        ],
        out_specs=pl.BlockSpec((tb, Vp), lambda i: (i, 0)),
        scratch_shapes=[pltpu.VMEM((Vp, Kp), jnp.bfloat16)],
        compiler_params=pltpu.CompilerParams(
            dimension_semantics=("arbitrary",)),
    )(x, weight, bm)

    if Bp != B or Vp != V:
        out = out[:B, :V]
    return out.astype(out_dtype)
